# baseline (speedup 1.0000x reference)
import functools

import jax
import jax.numpy as jnp
from jax import lax
from jax.experimental import pallas as pl
from jax.experimental.pallas import tpu as pltpu
from jax.experimental.pallas import tpu_sc as plsc

D_MODEL = 64
NW = 32
NBUF = 4
L = 16


def _embed_body(xt_hbm, table_hbm, out_hbm, idx_v, idx2, rows_v, *sems):
    wid = lax.axis_index("s") * 2 + lax.axis_index("c")
    steps = idx2.shape[0]
    n_cols = out_hbm.shape[1]
    base = wid * steps

    pltpu.sync_copy(
        xt_hbm.at[pl.ds(0, n_cols), pl.ds(base, steps)],
        idx_v.at[pl.ds(0, n_cols)],
    )

    def transpose_row(r, carry):
        for k in range(4):
            j_ids = lax.iota(jnp.int32, L) + (L * k)
            r_ids = jnp.full((L,), r, jnp.int32)
            idx2[r, pl.ds(L * k, L)] = plsc.load_gather(idx_v, [j_ids, r_ids])
        return carry

    lax.fori_loop(0, steps, transpose_row, 0)

    def gather(step, buf):
        return pltpu.async_copy(
            table_hbm.at[idx2.at[step].at[pl.ds(0, n_cols)]],
            rows_v.at[buf],
            sems[buf],
        )

    for b in range(NBUF):
        gather(b, b)

    def outer(o, carry):
        for b in range(NBUF):
            step = o * NBUF + b
            pltpu.make_async_copy(
                table_hbm.at[idx2.at[0].at[pl.ds(0, n_cols)]],
                rows_v.at[b],
                sems[b],
            ).wait()
            pltpu.sync_copy(rows_v.at[b], out_hbm.at[base + step])
            nxt = step + NBUF

            @pl.when(nxt < steps)
            def _():
                gather(nxt, b)

        return carry

    lax.fori_loop(0, steps // NBUF, outer, 0)


def kernel(x, table):
    n_rows, n_cols = x.shape
    xt = x.T.astype(jnp.int32)
    steps = n_rows // NW

    mesh = plsc.VectorSubcoreMesh(core_axis_name="c", subcore_axis_name="s")
    run = functools.partial(
        pl.kernel,
        mesh=mesh,
        compiler_params=pltpu.CompilerParams(
            use_tc_tiling_on_sc=False, needs_layout_passes=False
        ),
        out_type=jax.ShapeDtypeStruct((n_rows, n_cols, D_MODEL), jnp.float32),
        scratch_types=[
            pltpu.VMEM((64, steps), jnp.int32),
            pltpu.VMEM((steps, 64), jnp.int32),
            pltpu.VMEM((NBUF, n_cols, D_MODEL), jnp.float32),
        ]
        + [pltpu.SemaphoreType.DMA] * NBUF,
    )(_embed_body)

    return run(xt, table)


# speedup vs baseline: 1.0009x; 1.0009x over previous
import functools

import jax
import jax.numpy as jnp
from jax import lax
from jax.experimental import pallas as pl
from jax.experimental.pallas import tpu as pltpu
from jax.experimental.pallas import tpu_sc as plsc

D_MODEL = 64
NW = 32
NBUF = 4
L = 16


def _embed_body(xt_hbm, table_hbm, out_hbm, idx_v, idx2, rows_v, *sems):
    wid = lax.axis_index("s") * 2 + lax.axis_index("c")
    steps = idx2.shape[0]
    n_cols = out_hbm.shape[1]
    base = wid * steps

    pltpu.sync_copy(
        xt_hbm.at[pl.ds(0, n_cols), pl.ds(base, steps)],
        idx_v.at[pl.ds(0, n_cols)],
    )

    def transpose_row(r, carry):
        for k in range(4):
            j_ids = lax.iota(jnp.int32, L) + (L * k)
            r_ids = jnp.full((L,), r, jnp.int32)
            v = plsc.load_gather(idx_v, [j_ids, r_ids])
            idx2[r, pl.ds(L * k, L)] = plsc.bitcast(v, jnp.int32)
        return carry

    lax.fori_loop(0, steps, transpose_row, 0)

    def gather(step, buf):
        return pltpu.async_copy(
            table_hbm.at[idx2.at[step].at[pl.ds(0, n_cols)]],
            rows_v.at[buf],
            sems[buf],
        )

    for b in range(NBUF):
        gather(b, b)

    def outer(o, carry):
        for b in range(NBUF):
            step = o * NBUF + b
            pltpu.make_async_copy(
                table_hbm.at[idx2.at[0].at[pl.ds(0, n_cols)]],
                rows_v.at[b],
                sems[b],
            ).wait()
            pltpu.sync_copy(rows_v.at[b], out_hbm.at[base + step])
            nxt = step + NBUF

            @pl.when(nxt < steps)
            def _():
                gather(nxt, b)

        return carry

    lax.fori_loop(0, steps // NBUF, outer, 0)


def kernel(x, table):
    n_rows, n_cols = x.shape
    xt = lax.bitcast_convert_type(x.T.astype(jnp.int32), jnp.float32)
    steps = n_rows // NW

    mesh = plsc.VectorSubcoreMesh(core_axis_name="c", subcore_axis_name="s")
    run = functools.partial(
        pl.kernel,
        mesh=mesh,
        compiler_params=pltpu.CompilerParams(
            use_tc_tiling_on_sc=False, needs_layout_passes=False
        ),
        out_type=jax.ShapeDtypeStruct((n_rows, n_cols, D_MODEL), jnp.float32),
        scratch_types=[
            pltpu.VMEM((64, steps), jnp.float32),
            pltpu.VMEM((steps, 64), jnp.int32),
            pltpu.VMEM((NBUF, n_cols, D_MODEL), jnp.float32),
        ]
        + [pltpu.SemaphoreType.DMA] * NBUF,
    )(_embed_body)

    return run(xt, table)


# speedup vs baseline: 1.0149x; 1.0140x over previous
import functools

import jax
import jax.numpy as jnp
from jax import lax
from jax.experimental import pallas as pl
from jax.experimental.pallas import tpu as pltpu
from jax.experimental.pallas import tpu_sc as plsc

D_MODEL = 64
NW = 32
NBUF = 4


def _embed_body(xp_hbm, table_hbm, out_hbm, idx_v, rows_v, *sems):
    wid = lax.axis_index("s") * 2 + lax.axis_index("c")
    steps = idx_v.shape[0]
    n_cols = out_hbm.shape[1]
    base = wid * steps

    pltpu.sync_copy(xp_hbm.at[pl.ds(base, steps)], idx_v)

    def gather(step, buf):
        return pltpu.async_copy(
            table_hbm.at[idx_v.at[step].at[pl.ds(0, n_cols)]],
            rows_v.at[buf],
            sems[buf],
        )

    for b in range(NBUF):
        gather(b, b)

    def outer(o, carry):
        for b in range(NBUF):
            step = o * NBUF + b
            pltpu.make_async_copy(
                table_hbm.at[idx_v.at[0].at[pl.ds(0, n_cols)]],
                rows_v.at[b],
                sems[b],
            ).wait()
            pltpu.sync_copy(rows_v.at[b], out_hbm.at[base + step])
            nxt = step + NBUF

            @pl.when(nxt < steps)
            def _():
                gather(nxt, b)

        return carry

    lax.fori_loop(0, steps // NBUF, outer, 0)


def kernel(x, table):
    n_rows, n_cols = x.shape
    xi = x.astype(jnp.int32)
    xp = jnp.pad(xi, ((0, 0), (0, 128 - n_cols)))
    steps = n_rows // NW

    mesh = plsc.VectorSubcoreMesh(core_axis_name="c", subcore_axis_name="s")
    run = functools.partial(
        pl.kernel,
        mesh=mesh,
        compiler_params=pltpu.CompilerParams(use_tc_tiling_on_sc=False),
        out_type=jax.ShapeDtypeStruct((n_rows, n_cols, D_MODEL), jnp.float32),
        scratch_types=[
            pltpu.VMEM((steps, 128), jnp.int32),
            pltpu.VMEM((NBUF, n_cols, D_MODEL), jnp.float32),
        ]
        + [pltpu.SemaphoreType.DMA] * NBUF,
    )(_embed_body)

    return run(xp, table)
